# baseline (speedup 1.0000x reference)
import functools

import jax
import jax.numpy as jnp
from jax import lax
from jax.experimental import pallas as pl
from jax.experimental.pallas import tpu as pltpu
from jax.experimental.pallas import tpu_sc as plsc

NUM_PIXELS = 1000 * 1000
C = 64
EMBED_DIM = 128
N_IDX = 16384

_info = plsc.get_sparse_core_info()
NC, NS = _info.num_cores, _info.num_subcores
NW = NC * NS
BPW = N_IDX // NW
Q = BPW // 4
NJOBS = 8


def _sc_gather(table3, o_idx, d_idx):
    mesh = plsc.VectorSubcoreMesh(core_axis_name="c", subcore_axis_name="s")

    @functools.partial(
        pl.kernel,
        mesh=mesh,
        out_type=[
            jax.ShapeDtypeStruct((N_IDX, C), jnp.float32),
            jax.ShapeDtypeStruct((N_IDX, C), jnp.float32),
        ],
        scratch_types=[
            pltpu.VMEM((BPW,), jnp.int32),
            pltpu.VMEM((BPW,), jnp.int32),
            pltpu.VMEM((Q // 8, 8, 2 * C), jnp.float32),
            pltpu.VMEM((Q // 8, 8, 2 * C), jnp.float32),
            pltpu.VMEM((Q, C), jnp.float32),
            pltpu.VMEM((Q, C), jnp.float32),
            pltpu.SemaphoreType.DMA,
            pltpu.SemaphoreType.DMA,
        ],
    )
    def k(table_h, o_idx_h, d_idx_h, o_out_h, d_out_h,
          o_idx_s, d_idx_s, pair0, pair1, cbuf0, cbuf1, sem0, sem1):
        wid = lax.axis_index("s") * NC + lax.axis_index("c")
        base = wid * BPW
        pltpu.sync_copy(o_idx_h.at[pl.ds(base, BPW)], o_idx_s)
        pltpu.sync_copy(d_idx_h.at[pl.ds(base, BPW)], d_idx_s)

        pairs = (pair0, pair1)
        cbufs = (cbuf0, cbuf1)
        sems = (sem0, sem1)
        jobs = [(o_idx_s, o_out_h, q * Q) for q in range(4)]
        jobs += [(d_idx_s, d_out_h, q * Q) for q in range(4)]

        def fire(j, slot):
            idx_s, _, off = jobs[j]
            pair, sem = pairs[slot], sems[slot]

            def body(g, carry):
                vec = idx_s[pl.ds(off + g * 16, 16)]
                for lane in range(16):
                    p = vec[lane] // 2
                    row = g * 16 + lane
                    pltpu.async_copy(
                        table_h.at[p // 8, pl.ds(p % 8, 1)],
                        pair.at[row // 8, pl.ds(row % 8, 1)], sem)
                return carry

            lax.fori_loop(0, Q // 16, body, 0)

        def drain(slot):
            pltpu.make_async_copy(table_h.at[pl.ds(0, Q // 8)],
                                  pairs[slot], sems[slot]).wait()

        def compact(j, slot):
            idx_s, _, off = jobs[j]
            pair, cbuf = pairs[slot], cbufs[slot]

            def body(g, carry):
                vec = idx_s[pl.ds(off + g * 16, 16)]
                for lane in range(16):
                    r = vec[lane]
                    half = (r % 2) * C
                    row = g * 16 + lane
                    for cg in range(C // 16):
                        cbuf[row, pl.ds(cg * 16, 16)] = (
                            pair[row // 8, row % 8, pl.ds(half + cg * 16, 16)])
                return carry

            lax.fori_loop(0, Q // 16, body, 0)

        fire(0, 0)
        fire(1, 1)
        for j in range(NJOBS):
            slot = j % 2
            drain(slot)
            compact(j, slot)
            if j + 2 < NJOBS:
                fire(j + 2, slot)
            _, out_h, off = jobs[j]
            pltpu.sync_copy(cbufs[slot], out_h.at[pl.ds(base + off, Q)])

    return k(table3, o_idx, d_idx)


def _tc_mlp(o_rows, d_rows, W, b2):
    Br = 1024
    grid = (N_IDX // Br,)

    def body(o_ref, d_ref, w_ref, b_ref, oo_ref, do_ref):
        w = w_ref[...]
        bb = b_ref[...]
        oo_ref[...] = jnp.maximum(
            jnp.dot(o_ref[...], w, preferred_element_type=jnp.float32) + bb, 0.0)
        do_ref[...] = jnp.maximum(
            jnp.dot(d_ref[...], w, preferred_element_type=jnp.float32) + bb, 0.0)

    return pl.pallas_call(
        body,
        grid=grid,
        in_specs=[
            pl.BlockSpec((Br, C), lambda i: (i, 0)),
            pl.BlockSpec((Br, C), lambda i: (i, 0)),
            pl.BlockSpec((C, EMBED_DIM), lambda i: (0, 0)),
            pl.BlockSpec((1, EMBED_DIM), lambda i: (0, 0)),
        ],
        out_specs=[
            pl.BlockSpec((Br, EMBED_DIM), lambda i: (i, 0)),
            pl.BlockSpec((Br, EMBED_DIM), lambda i: (i, 0)),
        ],
        out_shape=[
            jax.ShapeDtypeStruct((N_IDX, EMBED_DIM), jnp.float32),
            jax.ShapeDtypeStruct((N_IDX, EMBED_DIM), jnp.float32),
        ],
    )(o_rows, d_rows, W, b2)


@jax.jit
def _impl(grid_features, W, b, o_indices, d_indices):
    table3 = grid_features.reshape(NUM_PIXELS // 16, 8, 2 * C)
    o_rows, d_rows = _sc_gather(table3, o_indices, d_indices)
    return _tc_mlp(o_rows, d_rows, W, b.reshape(1, EMBED_DIM))


def kernel(grid_features, W, b, o_indices, d_indices):
    return _impl(grid_features, W, b, o_indices, d_indices)


# speedup vs baseline: 2.4778x; 2.4778x over previous
import functools

import jax
import jax.numpy as jnp
from jax import lax
from jax.experimental import pallas as pl
from jax.experimental.pallas import tpu as pltpu
from jax.experimental.pallas import tpu_sc as plsc

NUM_PIXELS = 1000 * 1000
C = 64
EMBED_DIM = 128
N_IDX = 16384

_info = plsc.get_sparse_core_info()
NC, NS = _info.num_cores, _info.num_subcores
NW = NC * NS
BPW = N_IDX // NW
HALF = BPW // 2


def _sc_gather(table3, o_idx, d_idx):
    mesh = plsc.VectorSubcoreMesh(core_axis_name="c", subcore_axis_name="s")

    @functools.partial(
        pl.kernel,
        mesh=mesh,
        out_type=[
            jax.ShapeDtypeStruct((N_IDX, C), jnp.float32),
            jax.ShapeDtypeStruct((N_IDX, C), jnp.float32),
        ],
        scratch_types=[
            pltpu.VMEM((BPW,), jnp.int32),
            pltpu.VMEM((BPW,), jnp.int32),
            pltpu.VMEM((HALF, C), jnp.float32),
            pltpu.VMEM((HALF, C), jnp.float32),
            pltpu.SemaphoreType.DMA,
            pltpu.SemaphoreType.DMA,
        ],
    )
    def k(table_h, o_idx_h, d_idx_h, o_out_h, d_out_h,
          o_idx_s, d_idx_s, buf0, buf1, sem0, sem1):
        wid = lax.axis_index("s") * NC + lax.axis_index("c")
        base = wid * BPW
        pltpu.sync_copy(o_idx_h.at[pl.ds(base, BPW)], o_idx_s)
        pltpu.sync_copy(d_idx_h.at[pl.ds(base, BPW)], d_idx_s)

        bufs = (buf0, buf1)
        sems = (sem0, sem1)
        jobs = [(o_idx_s, o_out_h, 0), (o_idx_s, o_out_h, HALF),
                (d_idx_s, d_out_h, 0), (d_idx_s, d_out_h, HALF)]

        def fire(j, slot):
            idx_s, _, off = jobs[j]
            buf, sem = bufs[slot], sems[slot]

            def body(g, carry):
                vec = idx_s[pl.ds(off + g * 16, 16)]
                for lane in range(16):
                    row = vec[lane]
                    pltpu.async_copy(
                        table_h.at[row // 8, pl.ds(row % 8, 1)],
                        buf.at[pl.ds(g * 16 + lane, 1)], sem)
                return carry

            lax.fori_loop(0, HALF // 16, body, 0)

        def drain(slot):
            pltpu.make_async_copy(o_out_h.at[pl.ds(0, HALF)],
                                  bufs[slot], sems[slot]).wait()

        fire(0, 0)
        fire(1, 1)
        for j in range(len(jobs)):
            slot = j % 2
            drain(slot)
            _, out_h, off = jobs[j]
            pltpu.sync_copy(bufs[slot], out_h.at[pl.ds(base + off, HALF)])
            if j + 2 < len(jobs):
                fire(j + 2, slot)

    return k(table3, o_idx, d_idx)


def _tc_mlp(o_rows, d_rows, W, b2):
    Br = 1024
    grid = (N_IDX // Br,)

    def body(o_ref, d_ref, w_ref, b_ref, oo_ref, do_ref):
        w = w_ref[...]
        bb = b_ref[...]
        oo_ref[...] = jnp.maximum(
            jnp.dot(o_ref[...], w, preferred_element_type=jnp.float32) + bb, 0.0)
        do_ref[...] = jnp.maximum(
            jnp.dot(d_ref[...], w, preferred_element_type=jnp.float32) + bb, 0.0)

    return pl.pallas_call(
        body,
        grid=grid,
        in_specs=[
            pl.BlockSpec((Br, C), lambda i: (i, 0)),
            pl.BlockSpec((Br, C), lambda i: (i, 0)),
            pl.BlockSpec((C, EMBED_DIM), lambda i: (0, 0)),
            pl.BlockSpec((1, EMBED_DIM), lambda i: (0, 0)),
        ],
        out_specs=[
            pl.BlockSpec((Br, EMBED_DIM), lambda i: (i, 0)),
            pl.BlockSpec((Br, EMBED_DIM), lambda i: (i, 0)),
        ],
        out_shape=[
            jax.ShapeDtypeStruct((N_IDX, EMBED_DIM), jnp.float32),
            jax.ShapeDtypeStruct((N_IDX, EMBED_DIM), jnp.float32),
        ],
    )(o_rows, d_rows, W, b2)


@jax.jit
def _impl(grid_features, W, b, o_indices, d_indices):
    table3 = grid_features.reshape(NUM_PIXELS // 8, 8, C)
    o_rows, d_rows = _sc_gather(table3, o_indices, d_indices)
    return _tc_mlp(o_rows, d_rows, W, b.reshape(1, EMBED_DIM))


def kernel(grid_features, W, b, o_indices, d_indices):
    return _impl(grid_features, W, b, o_indices, d_indices)
